# Initial kernel scaffold; baseline (speedup 1.0000x reference)
import jax
import jax.numpy as jnp
from jax.experimental import pallas as pl


def kernel(fasta_sequence, out, W_i, W_j, W_rel):
    raise NotImplementedError("write your pallas kernel here")


# speedup vs baseline: 9.5316x; 9.5316x over previous
import functools

import jax
import jax.numpy as jnp
from jax.experimental import pallas as pl
from jax.experimental.pallas import tpu as pltpu

_L = 512
_D = 128
_BI = 8
_NG = 2 * _L + 1024


def _edge_body(idx_ref, tab_ref, x_ref, o_ref, g_ref):
    @pl.when(pl.program_id(0) == 0)
    def _build_g():
        idx = idx_ref[...]
        onehot = (jax.lax.broadcasted_iota(jnp.int32, (_NG, 128), 1)
                  == idx).astype(jnp.float32)
        g_ref[...] = jax.lax.dot_general(
            onehot, tab_ref[...], (((1,), (0,)), ((), ())),
            preferred_element_type=jnp.float32)

    i0 = pl.program_id(0) * _BI
    pj = g_ref[_L:2 * _L, :]
    for r in range(_BI):
        pi = g_ref[pl.ds(i0 + r, 1), :]
        rel = g_ref[pl.ds(2 * _L + _L - 1 - (i0 + r), _L), :]
        o_ref[r] = x_ref[r] + pi + pj + rel


@functools.partial(jax.jit, static_argnames=())
def kernel(fasta_sequence, out, W_i, W_j, W_rel):
    seq = fasta_sequence.reshape(_L).astype(jnp.int32)
    n_i = W_i.shape[0]
    n_rel = W_rel.shape[0]
    one_side = n_rel // 2

    k = jnp.arange(1024, dtype=jnp.int32)
    rel_idx = jnp.clip(k - (_L - 1), -one_side, one_side) + one_side
    idx_all = jnp.concatenate(
        [seq, seq + n_i, rel_idx + n_i + W_j.shape[0]]).reshape(_NG, 1)

    tab = jnp.concatenate([W_i, W_j, W_rel], axis=0)
    tab = jnp.pad(tab, ((0, 128 - tab.shape[0]), (0, 0)))

    x = out.reshape(_L, _L, _D)
    res = pl.pallas_call(
        _edge_body,
        grid=(_L // _BI,),
        in_specs=[
            pl.BlockSpec((_NG, 1), lambda i: (0, 0)),
            pl.BlockSpec((128, _D), lambda i: (0, 0)),
            pl.BlockSpec((_BI, _L, _D), lambda i: (i, 0, 0)),
        ],
        out_specs=pl.BlockSpec((_BI, _L, _D), lambda i: (i, 0, 0)),
        out_shape=jax.ShapeDtypeStruct((_L, _L, _D), jnp.float32),
        scratch_shapes=[pltpu.VMEM((_NG, _D), jnp.float32)],
    )(idx_all, tab, x)
    return res.reshape(out.shape)


# speedup vs baseline: 10.7940x; 1.1324x over previous
import functools

import jax
import jax.numpy as jnp
from jax.experimental import pallas as pl
from jax.experimental.pallas import tpu as pltpu

_L = 512
_D = 128
_BI = 16
_NG = 2 * _L + 1024


def _edge_body(idx_ref, tab_ref, x_ref, o_ref, g_ref):
    @pl.when(pl.program_id(0) == 0)
    def _build_g():
        idx = idx_ref[...]
        onehot = (jax.lax.broadcasted_iota(jnp.int32, (_NG, 128), 1)
                  == idx).astype(jnp.float32)
        g_ref[...] = jax.lax.dot_general(
            onehot, tab_ref[...], (((1,), (0,)), ((), ())),
            preferred_element_type=jnp.float32)

    i0 = pl.program_id(0) * _BI
    pj = g_ref[_L:2 * _L, :]
    for r in range(_BI):
        pi = g_ref[pl.ds(i0 + r, 1), :]
        rel = g_ref[pl.ds(2 * _L + _L - 1 - (i0 + r), _L), :]
        o_ref[r] = x_ref[r] + pi + pj + rel


@functools.partial(jax.jit, static_argnames=())
def kernel(fasta_sequence, out, W_i, W_j, W_rel):
    seq = fasta_sequence.reshape(_L).astype(jnp.int32)
    n_i = W_i.shape[0]
    n_rel = W_rel.shape[0]
    one_side = n_rel // 2

    k = jnp.arange(1024, dtype=jnp.int32)
    rel_idx = jnp.clip(k - (_L - 1), -one_side, one_side) + one_side
    idx_all = jnp.concatenate(
        [seq, seq + n_i, rel_idx + n_i + W_j.shape[0]]).reshape(_NG, 1)

    tab = jnp.concatenate([W_i, W_j, W_rel], axis=0)
    tab = jnp.pad(tab, ((0, 128 - tab.shape[0]), (0, 0)))

    x = out.reshape(_L, _L, _D)
    res = pl.pallas_call(
        _edge_body,
        grid=(_L // _BI,),
        in_specs=[
            pl.BlockSpec((_NG, 1), lambda i: (0, 0)),
            pl.BlockSpec((128, _D), lambda i: (0, 0)),
            pl.BlockSpec((_BI, _L, _D), lambda i: (i, 0, 0)),
        ],
        out_specs=pl.BlockSpec((_BI, _L, _D), lambda i: (i, 0, 0)),
        out_shape=jax.ShapeDtypeStruct((_L, _L, _D), jnp.float32),
        scratch_shapes=[pltpu.VMEM((_NG, _D), jnp.float32)],
    )(idx_all, tab, x)
    return res.reshape(out.shape)


# speedup vs baseline: 11.0748x; 1.0260x over previous
import functools

import jax
import jax.numpy as jnp
from jax.experimental import pallas as pl
from jax.experimental.pallas import tpu as pltpu

_L = 512
_D = 128
_BI = 32
_NG = 2 * _L + 1024


def _edge_body(idx_ref, tab_ref, x_ref, o_ref, g_ref):
    @pl.when(pl.program_id(0) == 0)
    def _build_g():
        idx = idx_ref[...]
        onehot = (jax.lax.broadcasted_iota(jnp.int32, (_NG, 128), 1)
                  == idx).astype(jnp.float32)
        g_ref[...] = jax.lax.dot_general(
            onehot, tab_ref[...], (((1,), (0,)), ((), ())),
            preferred_element_type=jnp.float32)

    i0 = pl.program_id(0) * _BI
    pj = g_ref[_L:2 * _L, :]
    for r in range(_BI):
        pi = g_ref[pl.ds(i0 + r, 1), :]
        rel = g_ref[pl.ds(2 * _L + _L - 1 - (i0 + r), _L), :]
        o_ref[r] = x_ref[r] + pi + pj + rel


@functools.partial(jax.jit, static_argnames=())
def kernel(fasta_sequence, out, W_i, W_j, W_rel):
    seq = fasta_sequence.reshape(_L).astype(jnp.int32)
    n_i = W_i.shape[0]
    n_rel = W_rel.shape[0]
    one_side = n_rel // 2

    k = jnp.arange(1024, dtype=jnp.int32)
    rel_idx = jnp.clip(k - (_L - 1), -one_side, one_side) + one_side
    idx_all = jnp.concatenate(
        [seq, seq + n_i, rel_idx + n_i + W_j.shape[0]]).reshape(_NG, 1)

    tab = jnp.concatenate([W_i, W_j, W_rel], axis=0)
    tab = jnp.pad(tab, ((0, 128 - tab.shape[0]), (0, 0)))

    x = out.reshape(_L, _L, _D)
    res = pl.pallas_call(
        _edge_body,
        grid=(_L // _BI,),
        in_specs=[
            pl.BlockSpec((_NG, 1), lambda i: (0, 0)),
            pl.BlockSpec((128, _D), lambda i: (0, 0)),
            pl.BlockSpec((_BI, _L, _D), lambda i: (i, 0, 0)),
        ],
        out_specs=pl.BlockSpec((_BI, _L, _D), lambda i: (i, 0, 0)),
        out_shape=jax.ShapeDtypeStruct((_L, _L, _D), jnp.float32),
        scratch_shapes=[pltpu.VMEM((_NG, _D), jnp.float32)],
    )(idx_all, tab, x)
    return res.reshape(out.shape)
